# baseline (speedup 1.0000x reference)
import functools

import jax
import jax.numpy as jnp
from jax import lax
from jax.experimental import pallas as pl
from jax.experimental.pallas import tpu as pltpu
from jax.experimental.pallas import tpu_sc as plsc

_N = 10000
_NP = 10240
_E = 320000
_D = 128
_H1 = 50
_HP1 = 64
_H2 = 2
_HP2 = 16
_C = 10
_DW = 8

_NC = 2
_NS = 16
_NW = _NC * _NS
_EP = 327680
_EB = 128
_EC = 80
_RPS = _NP // _NS
_ZB = 32
_ZC = _RPS // _ZB
_NB = 8
_NG = _EC // _NB

_mesh = functools.partial(
    plsc.VectorSubcoreMesh, core_axis_name="c", subcore_axis_name="s"
)


def _deg_body(edge_hbm, ones_hbm, zrows_hbm, out_hbm, dst_v, ones_v, zrows_v,
              acc_sh, sem):
    cid = lax.axis_index("c")
    sid = lax.axis_index("s")
    wid = sid * _NC + cid
    base = sid * _RPS
    pltpu.sync_copy(zrows_hbm, zrows_v)
    for t in range(_ZC):
        pltpu.sync_copy(zrows_v, acc_sh.at[pl.ds(base + t * _ZB, _ZB)])
    pltpu.sync_copy(ones_hbm, ones_v)
    pltpu.sync_copy(edge_hbm.at[0, pl.ds(wid * _EC, _EC)], dst_v)
    plsc.subcore_barrier()

    def group(g, carry):
        descs = []
        for b in range(_NB):
            descs.append(
                pltpu.async_copy(ones_v, acc_sh.at[dst_v.at[g * _NB + b]],
                                 sem.at[b], add=True))
        for d in descs:
            d.wait()
        return carry

    lax.fori_loop(0, _NG, group, 0)
    plsc.subcore_barrier()
    pltpu.sync_copy(acc_sh.at[pl.ds(base, _RPS)],
                    out_hbm.at[cid, pl.ds(base, _RPS)])


_deg = pl.kernel(
    _deg_body,
    out_type=jax.ShapeDtypeStruct((_NC, _NP, _DW), jnp.float32),
    mesh=_mesh(),
    scratch_types=[
        pltpu.VMEM((_EC, _EB), jnp.int32),
        pltpu.VMEM((_EB, _DW), jnp.float32),
        pltpu.VMEM((_ZB, _DW), jnp.float32),
        pltpu.VMEM_SHARED((_NP, _DW), jnp.float32),
        pltpu.SemaphoreType.DMA((_NB,)),
    ],
    compiler_params=pltpu.CompilerParams(use_tc_tiling_on_sc=False),
)


def _make_agg(width, srow, drow):
    def body(y_hbm, edge_hbm, zrows_hbm, out_hbm,
             src_v, dst_v, rows_v, zrows_v, acc_sh, sem_g, sem_s):
        cid = lax.axis_index("c")
        sid = lax.axis_index("s")
        wid = sid * _NC + cid
        base = sid * _RPS
        pltpu.sync_copy(zrows_hbm, zrows_v)
        for t in range(_ZC):
            pltpu.sync_copy(zrows_v, acc_sh.at[pl.ds(base + t * _ZB, _ZB)])
        pltpu.sync_copy(edge_hbm.at[srow, pl.ds(wid * _EC, _EC)], src_v)
        pltpu.sync_copy(edge_hbm.at[drow, pl.ds(wid * _EC, _EC)], dst_v)
        plsc.subcore_barrier()

        def group(g, carry):
            gds = []
            for b in range(_NB):
                j = g * _NB + b

                @pl.when(g > 0)
                def _wait_prev_scatter():
                    pltpu.make_async_copy(
                        rows_v.at[b], acc_sh.at[dst_v.at[j - _NB]],
                        sem_s.at[b]).wait()

                gds.append(
                    pltpu.async_copy(y_hbm.at[src_v.at[j]], rows_v.at[b],
                                     sem_g.at[b]))
            for b in range(_NB):
                j = g * _NB + b
                gds[b].wait()
                pltpu.async_copy(rows_v.at[b], acc_sh.at[dst_v.at[j]],
                                 sem_s.at[b], add=True)
            return carry

        lax.fori_loop(0, _NG, group, 0)
        for b in range(_NB):
            j = (_NG - 1) * _NB + b
            pltpu.make_async_copy(
                rows_v.at[b], acc_sh.at[dst_v.at[j]], sem_s.at[b]).wait()
        plsc.subcore_barrier()
        pltpu.sync_copy(acc_sh.at[pl.ds(base, _RPS)],
                        out_hbm.at[cid, pl.ds(base, _RPS)])

    return pl.kernel(
        body,
        out_type=jax.ShapeDtypeStruct((_NC, _NP, width), jnp.float32),
        mesh=_mesh(),
        scratch_types=[
            pltpu.VMEM((_EC, _EB), jnp.int32),
            pltpu.VMEM((_EC, _EB), jnp.int32),
            pltpu.VMEM((_NB, _EB, width), jnp.float32),
            pltpu.VMEM((_ZB, width), jnp.float32),
            pltpu.VMEM_SHARED((_NP, width), jnp.float32),
            pltpu.SemaphoreType.DMA((_NB,)),
            pltpu.SemaphoreType.DMA((_NB,)),
        ],
        compiler_params=pltpu.CompilerParams(use_tc_tiling_on_sc=False),
    )


_agg1 = _make_agg(_HP1, 1, 2)
_agg2 = _make_agg(_HP2, 3, 4)

_NH = _NP * _HP1 // 128
_NQ = _NP * _HP2 // 128
_HB = _NP // 2
_QB = _NP // 8


_RA = 1024


def _tc_a0_body(xa_ref, xb_ref, w_ref, xw_ref):
    xwa = jnp.dot(xa_ref[...], w_ref[...], preferred_element_type=jnp.float32)
    xwb = jnp.dot(xb_ref[...], w_ref[...], preferred_element_type=jnp.float32)
    xw_ref[...] = jnp.concatenate([xwa, xwb], axis=1)


_tc_a0 = pl.pallas_call(
    _tc_a0_body,
    grid=(_NH // _RA,),
    in_specs=[
        pl.BlockSpec((_RA, _D), lambda i: (i, 0)),
        pl.BlockSpec((_RA, _D), lambda i: (5 + i, 0)),
        pl.BlockSpec((_D, _HP1), lambda i: (0, 0)),
    ],
    out_specs=pl.BlockSpec((_RA, 128), lambda i: (i, 0)),
    out_shape=jax.ShapeDtypeStruct((_NH, 128), jnp.float32),
)


def _dinv_of(ref):
    deg = ref[0] + ref[1] + 1.0
    return lax.rsqrt(jnp.maximum(deg[:, 0:1], 1.0))


def _tc_da_body(da_ref, db_ref, d0_ref, d1_ref, d2_ref, d3_ref, d4_ref,
                d5_ref, d6_ref, d7_ref, dinv2_ref, dinv16_ref):
    dinv2_ref[...] = jnp.concatenate(
        [jnp.broadcast_to(_dinv_of(da_ref), (_RA, _HP1)),
         jnp.broadcast_to(_dinv_of(db_ref), (_RA, _HP1))], axis=1)
    parts = [jnp.broadcast_to(_dinv_of(r), (_RA // 4, _HP2))
             for r in (d0_ref, d1_ref, d2_ref, d3_ref,
                       d4_ref, d5_ref, d6_ref, d7_ref)]
    dinv16_ref[...] = jnp.concatenate(parts, axis=1)


_tc_da = pl.pallas_call(
    _tc_da_body,
    grid=(_NH // _RA,),
    in_specs=[
        pl.BlockSpec((_NC, _RA, _DW), lambda i: (0, i, 0)),
        pl.BlockSpec((_NC, _RA, _DW), lambda i: (0, 5 + i, 0)),
    ] + [
        pl.BlockSpec((_NC, _RA // 4, _DW),
                     lambda i, k=k: (0, 5 * k + i, 0))
        for k in range(8)
    ],
    out_specs=[
        pl.BlockSpec((_RA, 128), lambda i: (i, 0)),
        pl.BlockSpec((_RA // 4, 128), lambda i: (i, 0)),
    ],
    out_shape=[
        jax.ShapeDtypeStruct((_NH, 128), jnp.float32),
        jax.ShapeDtypeStruct((_NQ, 128), jnp.float32),
    ],
)


def _tc_a1_body(xw_ref, dinv2_ref, y1_ref):
    y1_ref[...] = xw_ref[...] * dinv2_ref[...]


_tc_a1 = pl.pallas_call(
    _tc_a1_body,
    grid=(_NH // _RA,),
    in_specs=[
        pl.BlockSpec((_RA, 128), lambda i: (i, 0)),
        pl.BlockSpec((_RA, 128), lambda i: (i, 0)),
    ],
    out_specs=pl.BlockSpec((_RA, 128), lambda i: (i, 0)),
    out_shape=jax.ShapeDtypeStruct((_NH, 128), jnp.float32),
)


def _tc_b_body(aggp_ref, y1_ref, dinv2_ref, dinv16_ref, b1_ref, w2_ref,
               y2_ref):
    s = (aggp_ref[0] + aggp_ref[1] + y1_ref[...]) * dinv2_ref[...] \
        + b1_ref[...]
    h1 = jnp.tanh(s)
    y2 = jnp.dot(h1, w2_ref[...], preferred_element_type=jnp.float32)
    cols = [y2[k * _QB:(k + 1) * _QB, 0:_HP2] for k in range(4)] + \
           [y2[k * _QB:(k + 1) * _QB, _HP2:2 * _HP2] for k in range(4)]
    y2_ref[...] = jnp.concatenate(cols, axis=1) * dinv16_ref[...]


_tc_b = pl.pallas_call(
    _tc_b_body,
    in_specs=[
        pl.BlockSpec((_NC, _NH, 128), lambda: (0, 0, 0)),
        pl.BlockSpec((_NH, 128), lambda: (0, 0)),
        pl.BlockSpec((_NH, 128), lambda: (0, 0)),
        pl.BlockSpec((_NQ, 128), lambda: (0, 0)),
        pl.BlockSpec((1, 128), lambda: (0, 0)),
        pl.BlockSpec((128, 2 * _HP2), lambda: (0, 0)),
    ],
    out_specs=pl.BlockSpec((_NQ, 128), lambda: (0, 0)),
    out_shape=jax.ShapeDtypeStruct((_NQ, 128), jnp.float32),
)


def _tc_c_body(aggp_ref, y2_ref, dinv16_ref, b2_ref, wc_ref, bc_ref,
               h_ref, out_ref):
    s = (aggp_ref[0] + aggp_ref[1] + y2_ref[...]) * dinv16_ref[...] \
        + b2_ref[...]
    hp = jnp.tanh(s)
    h = jnp.concatenate(
        [hp[:, 16 * k:16 * (k + 1)] for k in range(8)], axis=0)[0:_N]
    h_ref[...] = h[:, 0:_H2]
    out_ref[...] = (
        jnp.dot(h, wc_ref[...], preferred_element_type=jnp.float32)
        + bc_ref[...]
    )


_tc_c = pl.pallas_call(
    _tc_c_body,
    in_specs=[
        pl.BlockSpec((_NC, _NQ, 128), lambda: (0, 0, 0)),
        pl.BlockSpec((_NQ, 128), lambda: (0, 0)),
        pl.BlockSpec((_NQ, 128), lambda: (0, 0)),
        pl.BlockSpec((1, 128), lambda: (0, 0)),
        pl.BlockSpec((_HP2, _C), lambda: (0, 0)),
        pl.BlockSpec((1, _C), lambda: (0, 0)),
    ],
    out_specs=[
        pl.BlockSpec((_N, _H2), lambda: (0, 0)),
        pl.BlockSpec((_N, _C), lambda: (0, 0)),
    ],
    out_shape=[
        jax.ShapeDtypeStruct((_N, _H2), jnp.float32),
        jax.ShapeDtypeStruct((_N, _C), jnp.float32),
    ],
)


def kernel(x, edge_index, W1, b1, W2, b2, Wc, bc):
    npad = _EP - _E
    ar = jnp.arange(npad, dtype=jnp.int32)
    src0 = jnp.concatenate([edge_index[0], ar % _N])
    dst0 = jnp.concatenate([edge_index[1], _N + 16 + (ar % (_NP - _N - 16))])
    src1 = 2 * (src0 % _HB) + src0 // _HB
    dst1 = 2 * (dst0 % _HB) + dst0 // _HB
    src2 = 8 * (src0 % _QB) + src0 // _QB
    dst2 = 8 * (dst0 % _QB) + dst0 // _QB
    edges = jnp.stack([dst0, src1, dst1, src2, dst2]).reshape(
        5, _NW * _EC, _EB)

    xpad = jnp.pad(x, ((0, _NP - _N), (0, 0)))

    ones8 = jnp.ones((_EB, _DW), jnp.float32)
    z8 = jnp.zeros((_ZB, _DW), jnp.float32)
    zrows1 = jnp.zeros((_ZB, _HP1), jnp.float32)
    zrows2 = jnp.zeros((_ZB, _HP2), jnp.float32)

    W1p = jnp.pad(W1, ((0, 0), (0, _HP1 - _H1)))
    b1p = jnp.pad(b1, (0, _HP1 - _H1))
    b1pk = jnp.concatenate([b1p, b1p]).reshape(1, 128)
    b2p = jnp.pad(b2, (0, _HP2 - _H2))
    b2pk = jnp.tile(b2p, 8).reshape(1, 128)
    W2p = jnp.pad(W2, ((0, _HP1 - _H1), (0, _HP2 - _H2)))
    W2bd = jnp.zeros((128, 2 * _HP2), jnp.float32)
    W2bd = W2bd.at[:_HP1, :_HP2].set(W2p).at[_HP1:, _HP2:].set(W2p)
    Wcp = jnp.pad(Wc, ((0, _HP2 - _H2), (0, 0)))
    bcp = bc.reshape(1, _C)

    degp = _deg(edges, ones8, z8)
    xw_pk = _tc_a0(xpad, xpad, W1p)
    dinv2, dinv16 = _tc_da(*([degp] * 10))
    y1_pk = _tc_a1(xw_pk, dinv2)
    y1 = y1_pk.reshape(_NP, _HP1)
    agg1 = _agg1(y1, edges, zrows1)
    agg1_pk = agg1.reshape(_NC, _NH, 128)
    y2_pk = _tc_b(agg1_pk, y1_pk, dinv2, dinv16, b1pk, W2bd)
    y2 = y2_pk.reshape(_NP, _HP2)
    agg2 = _agg2(y2, edges, zrows2)
    agg2_pk = agg2.reshape(_NC, _NQ, 128)
    h, out = _tc_c(agg2_pk, y2_pk, dinv16, b2pk, Wcp, bcp)
    return (out, h)


# speedup vs baseline: 1.2321x; 1.2321x over previous
import functools

import jax
import jax.numpy as jnp
from jax import lax
from jax.experimental import pallas as pl
from jax.experimental.pallas import tpu as pltpu
from jax.experimental.pallas import tpu_sc as plsc

_N = 10000
_NP = 10240
_E = 320000
_D = 128
_H1 = 50
_HP1 = 64
_H2 = 2
_HP2 = 16
_C = 10
_DW = 8

_NC = 2
_NS = 16
_NW = _NC * _NS
_EP = 327680
_EB = 128
_EC = 80
_RPS = _NP // _NS
_ZB = 32
_ZC = _RPS // _ZB
_NB = 8
_NG = _EC // _NB

_mesh = functools.partial(
    plsc.VectorSubcoreMesh, core_axis_name="c", subcore_axis_name="s"
)


def _deg_body(edge_hbm, ones_hbm, zrows_hbm, out_hbm, dst_v, ones_v, zrows_v,
              acc_sh, sem):
    cid = lax.axis_index("c")
    sid = lax.axis_index("s")
    wid = sid * _NC + cid
    base = sid * _RPS
    pltpu.sync_copy(zrows_hbm, zrows_v)
    for t in range(_ZC):
        pltpu.sync_copy(zrows_v, acc_sh.at[pl.ds(base + t * _ZB, _ZB)])
    pltpu.sync_copy(ones_hbm, ones_v)
    pltpu.sync_copy(edge_hbm.at[0, pl.ds(wid * _EC, _EC)], dst_v)
    plsc.subcore_barrier()

    def group(g, carry):
        descs = []
        for b in range(_NB):
            descs.append(
                pltpu.async_copy(ones_v, acc_sh.at[dst_v.at[g * _NB + b]],
                                 sem.at[b], add=True))
        for d in descs:
            d.wait()
        return carry

    lax.fori_loop(0, _NG, group, 0)
    plsc.subcore_barrier()
    pltpu.sync_copy(acc_sh.at[pl.ds(base, _RPS)],
                    out_hbm.at[cid, pl.ds(base, _RPS)])


_deg = pl.kernel(
    _deg_body,
    out_type=jax.ShapeDtypeStruct((_NC, _NP, _DW), jnp.float32),
    mesh=_mesh(),
    scratch_types=[
        pltpu.VMEM((_EC, _EB), jnp.int32),
        pltpu.VMEM((_EB, _DW), jnp.float32),
        pltpu.VMEM((_ZB, _DW), jnp.float32),
        pltpu.VMEM_SHARED((_NP, _DW), jnp.float32),
        pltpu.SemaphoreType.DMA((_NB,)),
    ],
    compiler_params=pltpu.CompilerParams(use_tc_tiling_on_sc=False),
)


def _make_agg(width, srow, drow):
    def body(y_hbm, edge_hbm, zrows_hbm, out_hbm,
             src_v, dst_v, rows_v, zrows_v, acc_sh, sem_g, sem_s):
        cid = lax.axis_index("c")
        sid = lax.axis_index("s")
        wid = sid * _NC + cid
        base = sid * _RPS
        pltpu.sync_copy(zrows_hbm, zrows_v)
        for t in range(_ZC):
            pltpu.sync_copy(zrows_v, acc_sh.at[pl.ds(base + t * _ZB, _ZB)])
        pltpu.sync_copy(edge_hbm.at[srow, pl.ds(wid * _EC, _EC)], src_v)
        pltpu.sync_copy(edge_hbm.at[drow, pl.ds(wid * _EC, _EC)], dst_v)
        plsc.subcore_barrier()

        def group(g, carry):
            gds = []
            for b in range(_NB):
                j = g * _NB + b

                @pl.when(g > 0)
                def _wait_prev_scatter():
                    pltpu.make_async_copy(
                        rows_v.at[b], acc_sh.at[dst_v.at[j - _NB]],
                        sem_s.at[b]).wait()

                gds.append(
                    pltpu.async_copy(y_hbm.at[src_v.at[j]], rows_v.at[b],
                                     sem_g.at[b]))
            for b in range(_NB):
                j = g * _NB + b
                gds[b].wait()
                pltpu.async_copy(rows_v.at[b], acc_sh.at[dst_v.at[j]],
                                 sem_s.at[b], add=True)
            return carry

        lax.fori_loop(0, _NG, group, 0)
        for b in range(_NB):
            j = (_NG - 1) * _NB + b
            pltpu.make_async_copy(
                rows_v.at[b], acc_sh.at[dst_v.at[j]], sem_s.at[b]).wait()
        plsc.subcore_barrier()
        pltpu.sync_copy(acc_sh.at[pl.ds(base, _RPS)],
                        out_hbm.at[cid, pl.ds(base, _RPS)])

    return pl.kernel(
        body,
        out_type=jax.ShapeDtypeStruct((_NC, _NP, width), jnp.float32),
        mesh=_mesh(),
        scratch_types=[
            pltpu.VMEM((_EC, _EB), jnp.int32),
            pltpu.VMEM((_EC, _EB), jnp.int32),
            pltpu.VMEM((_NB, _EB, width), jnp.float32),
            pltpu.VMEM((_ZB, width), jnp.float32),
            pltpu.VMEM_SHARED((_NP, width), jnp.float32),
            pltpu.SemaphoreType.DMA((_NB,)),
            pltpu.SemaphoreType.DMA((_NB,)),
        ],
        compiler_params=pltpu.CompilerParams(use_tc_tiling_on_sc=False),
    )


_agg1 = _make_agg(_HP1, 1, 2)
_agg2 = _make_agg(_HP2, 3, 4)

_NH = _NP * _HP1 // 128
_NQ = _NP * _HP2 // 128
_HB = _NP // 2
_QB = _NP // 8


_RA = 1024


def _tc_a0_body(xa_ref, xb_ref, w_ref, xw_ref):
    xwa = jnp.dot(xa_ref[...], w_ref[...], preferred_element_type=jnp.float32)
    xwb = jnp.dot(xb_ref[...], w_ref[...], preferred_element_type=jnp.float32)
    xw_ref[...] = jnp.concatenate([xwa, xwb], axis=1)


_tc_a0 = pl.pallas_call(
    _tc_a0_body,
    grid=(_NH // _RA,),
    in_specs=[
        pl.BlockSpec((_RA, _D), lambda i: (i, 0)),
        pl.BlockSpec((_RA, _D), lambda i: (5 + i, 0)),
        pl.BlockSpec((_D, _HP1), lambda i: (0, 0)),
    ],
    out_specs=pl.BlockSpec((_RA, 128), lambda i: (i, 0)),
    out_shape=jax.ShapeDtypeStruct((_NH, 128), jnp.float32),
)


def _dinv_of(ref):
    deg = ref[0] + ref[1] + 1.0
    return lax.rsqrt(jnp.maximum(deg[:, 0:1], 1.0))


def _tc_da_body(da_ref, db_ref, d0_ref, d1_ref, d2_ref, d3_ref, d4_ref,
                d5_ref, d6_ref, d7_ref, dinv2_ref, dinv16_ref):
    dinv2_ref[...] = jnp.concatenate(
        [jnp.broadcast_to(_dinv_of(da_ref), (_RA, _HP1)),
         jnp.broadcast_to(_dinv_of(db_ref), (_RA, _HP1))], axis=1)
    parts = [jnp.broadcast_to(_dinv_of(r), (_RA // 4, _HP2))
             for r in (d0_ref, d1_ref, d2_ref, d3_ref,
                       d4_ref, d5_ref, d6_ref, d7_ref)]
    dinv16_ref[...] = jnp.concatenate(parts, axis=1)


_tc_da = pl.pallas_call(
    _tc_da_body,
    grid=(_NH // _RA,),
    in_specs=[
        pl.BlockSpec((_NC, _RA, _DW), lambda i: (0, i, 0)),
        pl.BlockSpec((_NC, _RA, _DW), lambda i: (0, 5 + i, 0)),
    ] + [
        pl.BlockSpec((_NC, _RA // 4, _DW),
                     lambda i, k=k: (0, 5 * k + i, 0))
        for k in range(8)
    ],
    out_specs=[
        pl.BlockSpec((_RA, 128), lambda i: (i, 0)),
        pl.BlockSpec((_RA // 4, 128), lambda i: (i, 0)),
    ],
    out_shape=[
        jax.ShapeDtypeStruct((_NH, 128), jnp.float32),
        jax.ShapeDtypeStruct((_NQ, 128), jnp.float32),
    ],
)


def _tc_a1_body(xw_ref, dinv2_ref, y1_ref):
    y1_ref[...] = xw_ref[...] * dinv2_ref[...]


_tc_a1 = pl.pallas_call(
    _tc_a1_body,
    grid=(_NH // _RA,),
    in_specs=[
        pl.BlockSpec((_RA, 128), lambda i: (i, 0)),
        pl.BlockSpec((_RA, 128), lambda i: (i, 0)),
    ],
    out_specs=pl.BlockSpec((_RA, 128), lambda i: (i, 0)),
    out_shape=jax.ShapeDtypeStruct((_NH, 128), jnp.float32),
)


def _tc_b_body(aggp_ref, y1_ref, dinv2_ref, dinv16_ref, b1_ref, w2_ref,
               y2_ref):
    s = (aggp_ref[0] + aggp_ref[1] + y1_ref[...]) * dinv2_ref[...] \
        + b1_ref[...]
    h1 = jnp.tanh(s)
    y2 = jnp.dot(h1, w2_ref[...], preferred_element_type=jnp.float32)
    cols = [y2[k * _QB:(k + 1) * _QB, 0:_HP2] for k in range(4)] + \
           [y2[k * _QB:(k + 1) * _QB, _HP2:2 * _HP2] for k in range(4)]
    y2_ref[...] = jnp.concatenate(cols, axis=1) * dinv16_ref[...]


_tc_b = pl.pallas_call(
    _tc_b_body,
    in_specs=[
        pl.BlockSpec((_NC, _NH, 128), lambda: (0, 0, 0)),
        pl.BlockSpec((_NH, 128), lambda: (0, 0)),
        pl.BlockSpec((_NH, 128), lambda: (0, 0)),
        pl.BlockSpec((_NQ, 128), lambda: (0, 0)),
        pl.BlockSpec((1, 128), lambda: (0, 0)),
        pl.BlockSpec((128, 2 * _HP2), lambda: (0, 0)),
    ],
    out_specs=pl.BlockSpec((_NQ, 128), lambda: (0, 0)),
    out_shape=jax.ShapeDtypeStruct((_NQ, 128), jnp.float32),
)


def _tc_c_body(aggp_ref, y2_ref, dinv16_ref, b2_ref, wc_ref, bc_ref,
               h_ref, out_ref):
    s = (aggp_ref[0] + aggp_ref[1] + y2_ref[...]) * dinv16_ref[...] \
        + b2_ref[...]
    hp = jnp.tanh(s)
    h = jnp.concatenate(
        [hp[:, 16 * k:16 * (k + 1)] for k in range(8)], axis=0)[0:_N]
    h_ref[...] = h[:, 0:_H2]
    out_ref[...] = (
        jnp.dot(h, wc_ref[...], preferred_element_type=jnp.float32)
        + bc_ref[...]
    )


_tc_c = pl.pallas_call(
    _tc_c_body,
    in_specs=[
        pl.BlockSpec((_NC, _NQ, 128), lambda: (0, 0, 0)),
        pl.BlockSpec((_NQ, 128), lambda: (0, 0)),
        pl.BlockSpec((_NQ, 128), lambda: (0, 0)),
        pl.BlockSpec((1, 128), lambda: (0, 0)),
        pl.BlockSpec((_HP2, _C), lambda: (0, 0)),
        pl.BlockSpec((1, _C), lambda: (0, 0)),
    ],
    out_specs=[
        pl.BlockSpec((_N, _H2), lambda: (0, 0)),
        pl.BlockSpec((_N, _C), lambda: (0, 0)),
    ],
    out_shape=[
        jax.ShapeDtypeStruct((_N, _H2), jnp.float32),
        jax.ShapeDtypeStruct((_N, _C), jnp.float32),
    ],
)


def kernel(x, edge_index, W1, b1, W2, b2, Wc, bc):
    npad = _EP - _E
    ar = jnp.arange(npad, dtype=jnp.int32)
    src0 = jnp.concatenate([edge_index[0], ar])
    dst0 = jnp.concatenate([edge_index[1], _N + 16 + (ar & 127)])
    def perm1(v):
        hi = (v >= _HB).astype(jnp.int32)
        return 2 * (v - _HB * hi) + hi

    def perm2(v):
        k = ((v.astype(jnp.float32) + 0.5) * (1.0 / _QB)).astype(jnp.int32)
        return 8 * (v - _QB * k) + k

    src1 = perm1(src0)
    dst1 = perm1(dst0)
    src2 = perm2(src0)
    dst2 = perm2(dst0)
    edges = jnp.stack([dst0, src1, dst1, src2, dst2]).reshape(
        5, _NW * _EC, _EB)

    xpad = jnp.pad(x, ((0, _NP - _N), (0, 0)))

    ones8 = jnp.ones((_EB, _DW), jnp.float32)
    z8 = jnp.zeros((_ZB, _DW), jnp.float32)
    zrows1 = jnp.zeros((_ZB, _HP1), jnp.float32)
    zrows2 = jnp.zeros((_ZB, _HP2), jnp.float32)

    W1p = jnp.pad(W1, ((0, 0), (0, _HP1 - _H1)))
    b1p = jnp.pad(b1, (0, _HP1 - _H1))
    b1pk = jnp.concatenate([b1p, b1p]).reshape(1, 128)
    b2p = jnp.pad(b2, (0, _HP2 - _H2))
    b2pk = jnp.tile(b2p, 8).reshape(1, 128)
    W2p = jnp.pad(W2, ((0, _HP1 - _H1), (0, _HP2 - _H2)))
    W2bd = jnp.zeros((128, 2 * _HP2), jnp.float32)
    W2bd = W2bd.at[:_HP1, :_HP2].set(W2p).at[_HP1:, _HP2:].set(W2p)
    Wcp = jnp.pad(Wc, ((0, _HP2 - _H2), (0, 0)))
    bcp = bc.reshape(1, _C)

    degp = _deg(edges, ones8, z8)
    xw_pk = _tc_a0(xpad, xpad, W1p)
    dinv2, dinv16 = _tc_da(*([degp] * 10))
    y1_pk = _tc_a1(xw_pk, dinv2)
    y1 = y1_pk.reshape(_NP, _HP1)
    agg1 = _agg1(y1, edges, zrows1)
    agg1_pk = agg1.reshape(_NC, _NH, 128)
    y2_pk = _tc_b(agg1_pk, y1_pk, dinv2, dinv16, b1pk, W2bd)
    y2 = y2_pk.reshape(_NP, _HP2)
    agg2 = _agg2(y2, edges, zrows2)
    agg2_pk = agg2.reshape(_NC, _NQ, 128)
    h, out = _tc_c(agg2_pk, y2_pk, dinv16, b2pk, Wcp, bcp)
    return (out, h)


# speedup vs baseline: 1.2672x; 1.0285x over previous
import functools

import jax
import jax.numpy as jnp
from jax import lax
from jax.experimental import pallas as pl
from jax.experimental.pallas import tpu as pltpu
from jax.experimental.pallas import tpu_sc as plsc

_N = 10000
_NP = 10240
_E = 320000
_D = 128
_H1 = 50
_HP1 = 64
_H2 = 2
_HP2 = 16
_C = 10
_DW = 8

_NC = 2
_NS = 16
_NW = _NC * _NS
_EP = 327680
_EB = 128
_EC = 80
_RPS = _NP // _NS
_ZB = 32
_ZC = _RPS // _ZB
_NB = 8
_NG = _EC // _NB

_mesh = functools.partial(
    plsc.VectorSubcoreMesh, core_axis_name="c", subcore_axis_name="s"
)


def _deg_body(edge_hbm, ones_hbm, zrows_hbm, out_hbm, dst_v, ones_v, zrows_v,
              acc_sh, sem):
    cid = lax.axis_index("c")
    sid = lax.axis_index("s")
    wid = sid * _NC + cid
    base = sid * _RPS
    pltpu.sync_copy(zrows_hbm, zrows_v)
    for t in range(_ZC):
        pltpu.sync_copy(zrows_v, acc_sh.at[pl.ds(base + t * _ZB, _ZB)])
    pltpu.sync_copy(ones_hbm, ones_v)
    pltpu.sync_copy(edge_hbm.at[0, pl.ds(wid * _EC, _EC)], dst_v)
    plsc.subcore_barrier()

    def group(g, carry):
        descs = []
        for b in range(_NB):
            descs.append(
                pltpu.async_copy(ones_v, acc_sh.at[dst_v.at[g * _NB + b]],
                                 sem.at[b], add=True))
        for d in descs:
            d.wait()
        return carry

    lax.fori_loop(0, _NG, group, 0)
    plsc.subcore_barrier()
    pltpu.sync_copy(acc_sh.at[pl.ds(base, _RPS)],
                    out_hbm.at[cid, pl.ds(base, _RPS)])


_deg = pl.kernel(
    _deg_body,
    out_type=jax.ShapeDtypeStruct((_NC, _NP, _DW), jnp.float32),
    mesh=_mesh(),
    scratch_types=[
        pltpu.VMEM((_EC, _EB), jnp.int32),
        pltpu.VMEM((_EB, _DW), jnp.float32),
        pltpu.VMEM((_ZB, _DW), jnp.float32),
        pltpu.VMEM_SHARED((_NP, _DW), jnp.float32),
        pltpu.SemaphoreType.DMA((_NB,)),
    ],
    compiler_params=pltpu.CompilerParams(use_tc_tiling_on_sc=False),
)


def _make_agg(width, srow, drow):
    def body(y_hbm, edge_hbm, zrows_hbm, out_hbm,
             src_v, dst_v, rows_v, zrows_v, acc_sh, sem_g, sem_s):
        cid = lax.axis_index("c")
        sid = lax.axis_index("s")
        wid = sid * _NC + cid
        base = sid * _RPS
        pltpu.sync_copy(zrows_hbm, zrows_v)
        for t in range(_ZC):
            pltpu.sync_copy(zrows_v, acc_sh.at[pl.ds(base + t * _ZB, _ZB)])
        pltpu.sync_copy(edge_hbm.at[srow, pl.ds(wid * _EC, _EC)], src_v)
        pltpu.sync_copy(edge_hbm.at[drow, pl.ds(wid * _EC, _EC)], dst_v)
        plsc.subcore_barrier()

        def group(g, carry):
            gds = []
            for b in range(_NB):
                j = g * _NB + b

                @pl.when(g > 0)
                def _wait_prev_scatter():
                    pltpu.make_async_copy(
                        rows_v.at[b], acc_sh.at[dst_v.at[j - _NB]],
                        sem_s.at[b]).wait()

                gds.append(
                    pltpu.async_copy(y_hbm.at[src_v.at[j]], rows_v.at[b],
                                     sem_g.at[b]))
            for b in range(_NB):
                j = g * _NB + b
                gds[b].wait()
                pltpu.async_copy(rows_v.at[b], acc_sh.at[dst_v.at[j]],
                                 sem_s.at[b], add=True)
            return carry

        lax.fori_loop(0, _NG, group, 0)
        for b in range(_NB):
            j = (_NG - 1) * _NB + b
            pltpu.make_async_copy(
                rows_v.at[b], acc_sh.at[dst_v.at[j]], sem_s.at[b]).wait()
        plsc.subcore_barrier()
        pltpu.sync_copy(acc_sh.at[pl.ds(base, _RPS)],
                        out_hbm.at[cid, pl.ds(base, _RPS)])

    return pl.kernel(
        body,
        out_type=jax.ShapeDtypeStruct((_NC, _NP, width), jnp.float32),
        mesh=_mesh(),
        scratch_types=[
            pltpu.VMEM((_EC, _EB), jnp.int32),
            pltpu.VMEM((_EC, _EB), jnp.int32),
            pltpu.VMEM((_NB, _EB, width), jnp.float32),
            pltpu.VMEM((_ZB, width), jnp.float32),
            pltpu.VMEM_SHARED((_NP, width), jnp.float32),
            pltpu.SemaphoreType.DMA((_NB,)),
            pltpu.SemaphoreType.DMA((_NB,)),
        ],
        compiler_params=pltpu.CompilerParams(use_tc_tiling_on_sc=False),
    )


_agg1 = _make_agg(_HP1, 1, 2)
_agg2 = _make_agg(_HP2, 3, 4)

_NH = _NP * _HP1 // 128
_NQ = _NP * _HP2 // 128
_HB = _NP // 2
_QB = _NP // 8


_RA = 1024


def _tc_a0_body(xa_ref, xb_ref, w_ref, xw_ref):
    xwa = jnp.dot(xa_ref[...], w_ref[...], preferred_element_type=jnp.float32)
    xwb = jnp.dot(xb_ref[...], w_ref[...], preferred_element_type=jnp.float32)
    xw_ref[...] = jnp.concatenate([xwa, xwb], axis=1)


_tc_a0 = pl.pallas_call(
    _tc_a0_body,
    grid=(_NH // _RA,),
    in_specs=[
        pl.BlockSpec((_RA, _D), lambda i: (i, 0)),
        pl.BlockSpec((_RA, _D), lambda i: (5 + i, 0)),
        pl.BlockSpec((_D, _HP1), lambda i: (0, 0)),
    ],
    out_specs=pl.BlockSpec((_RA, 128), lambda i: (i, 0)),
    out_shape=jax.ShapeDtypeStruct((_NH, 128), jnp.float32),
)


def _dinv_of(ref):
    deg = ref[0] + ref[1] + 1.0
    return lax.rsqrt(jnp.maximum(deg[:, 0:1], 1.0))


def _tc_da_body(xw_ref, da_ref, db_ref, d0_ref, d1_ref, d2_ref, d3_ref,
                d4_ref, d5_ref, d6_ref, d7_ref, dinv2_ref, dinv16_ref,
                y1_ref):
    d2 = jnp.concatenate(
        [jnp.broadcast_to(_dinv_of(da_ref), (_RA, _HP1)),
         jnp.broadcast_to(_dinv_of(db_ref), (_RA, _HP1))], axis=1)
    dinv2_ref[...] = d2
    y1_ref[...] = xw_ref[...] * d2
    parts = [jnp.broadcast_to(_dinv_of(r), (_RA // 4, _HP2))
             for r in (d0_ref, d1_ref, d2_ref, d3_ref,
                       d4_ref, d5_ref, d6_ref, d7_ref)]
    dinv16_ref[...] = jnp.concatenate(parts, axis=1)


_tc_da = pl.pallas_call(
    _tc_da_body,
    grid=(_NH // _RA,),
    in_specs=[
        pl.BlockSpec((_RA, 128), lambda i: (i, 0)),
        pl.BlockSpec((_NC, _RA, _DW), lambda i: (0, i, 0)),
        pl.BlockSpec((_NC, _RA, _DW), lambda i: (0, 5 + i, 0)),
    ] + [
        pl.BlockSpec((_NC, _RA // 4, _DW),
                     lambda i, k=k: (0, 5 * k + i, 0))
        for k in range(8)
    ],
    out_specs=[
        pl.BlockSpec((_RA, 128), lambda i: (i, 0)),
        pl.BlockSpec((_RA // 4, 128), lambda i: (i, 0)),
        pl.BlockSpec((_RA, 128), lambda i: (i, 0)),
    ],
    out_shape=[
        jax.ShapeDtypeStruct((_NH, 128), jnp.float32),
        jax.ShapeDtypeStruct((_NQ, 128), jnp.float32),
        jax.ShapeDtypeStruct((_NH, 128), jnp.float32),
    ],
)


def _tc_b_body(aggp_ref, y1_ref, dinv2_ref, dinv16_ref, b1_ref, w2_ref,
               y2_ref):
    s = (aggp_ref[0] + aggp_ref[1] + y1_ref[...]) * dinv2_ref[...] \
        + b1_ref[...]
    h1 = jnp.tanh(s)
    y2 = jnp.dot(h1, w2_ref[...], preferred_element_type=jnp.float32)
    cols = [y2[k * _QB:(k + 1) * _QB, 0:_HP2] for k in range(4)] + \
           [y2[k * _QB:(k + 1) * _QB, _HP2:2 * _HP2] for k in range(4)]
    y2_ref[...] = jnp.concatenate(cols, axis=1) * dinv16_ref[...]


_tc_b = pl.pallas_call(
    _tc_b_body,
    in_specs=[
        pl.BlockSpec((_NC, _NH, 128), lambda: (0, 0, 0)),
        pl.BlockSpec((_NH, 128), lambda: (0, 0)),
        pl.BlockSpec((_NH, 128), lambda: (0, 0)),
        pl.BlockSpec((_NQ, 128), lambda: (0, 0)),
        pl.BlockSpec((1, 128), lambda: (0, 0)),
        pl.BlockSpec((128, 2 * _HP2), lambda: (0, 0)),
    ],
    out_specs=pl.BlockSpec((_NQ, 128), lambda: (0, 0)),
    out_shape=jax.ShapeDtypeStruct((_NQ, 128), jnp.float32),
)


def _tc_c_body(aggp_ref, y2_ref, dinv16_ref, b2_ref, wc_ref, bc_ref,
               h_ref, out_ref):
    s = (aggp_ref[0] + aggp_ref[1] + y2_ref[...]) * dinv16_ref[...] \
        + b2_ref[...]
    hp = jnp.tanh(s)
    h = jnp.concatenate(
        [hp[:, 16 * k:16 * (k + 1)] for k in range(8)], axis=0)[0:_N]
    h_ref[...] = h[:, 0:_H2]
    out_ref[...] = (
        jnp.dot(h, wc_ref[...], preferred_element_type=jnp.float32)
        + bc_ref[...]
    )


_tc_c = pl.pallas_call(
    _tc_c_body,
    in_specs=[
        pl.BlockSpec((_NC, _NQ, 128), lambda: (0, 0, 0)),
        pl.BlockSpec((_NQ, 128), lambda: (0, 0)),
        pl.BlockSpec((_NQ, 128), lambda: (0, 0)),
        pl.BlockSpec((1, 128), lambda: (0, 0)),
        pl.BlockSpec((_HP2, _C), lambda: (0, 0)),
        pl.BlockSpec((1, _C), lambda: (0, 0)),
    ],
    out_specs=[
        pl.BlockSpec((_N, _H2), lambda: (0, 0)),
        pl.BlockSpec((_N, _C), lambda: (0, 0)),
    ],
    out_shape=[
        jax.ShapeDtypeStruct((_N, _H2), jnp.float32),
        jax.ShapeDtypeStruct((_N, _C), jnp.float32),
    ],
)


def kernel(x, edge_index, W1, b1, W2, b2, Wc, bc):
    npad = _EP - _E
    ar = jnp.arange(npad, dtype=jnp.int32)
    src0 = jnp.concatenate([edge_index[0], ar])
    dst0 = jnp.concatenate([edge_index[1], _N + 16 + (ar & 127)])
    def perm1(v):
        hi = (v >= _HB).astype(jnp.int32)
        return 2 * (v - _HB * hi) + hi

    def perm2(v):
        k = ((v.astype(jnp.float32) + 0.5) * (1.0 / _QB)).astype(jnp.int32)
        return 8 * (v - _QB * k) + k

    src1 = perm1(src0)
    dst1 = perm1(dst0)
    src2 = perm2(src0)
    dst2 = perm2(dst0)
    edges = jnp.stack([dst0, src1, dst1, src2, dst2]).reshape(
        5, _NW * _EC, _EB)

    xpad = jnp.pad(x, ((0, _NP - _N), (0, 0)))

    ones8 = jnp.ones((_EB, _DW), jnp.float32)
    z8 = jnp.zeros((_ZB, _DW), jnp.float32)
    zrows1 = jnp.zeros((_ZB, _HP1), jnp.float32)
    zrows2 = jnp.zeros((_ZB, _HP2), jnp.float32)

    W1p = jnp.pad(W1, ((0, 0), (0, _HP1 - _H1)))
    b1p = jnp.pad(b1, (0, _HP1 - _H1))
    b1pk = jnp.concatenate([b1p, b1p]).reshape(1, 128)
    b2p = jnp.pad(b2, (0, _HP2 - _H2))
    b2pk = jnp.tile(b2p, 8).reshape(1, 128)
    W2p = jnp.pad(W2, ((0, _HP1 - _H1), (0, _HP2 - _H2)))
    W2bd = jnp.zeros((128, 2 * _HP2), jnp.float32)
    W2bd = W2bd.at[:_HP1, :_HP2].set(W2p).at[_HP1:, _HP2:].set(W2p)
    Wcp = jnp.pad(Wc, ((0, _HP2 - _H2), (0, 0)))
    bcp = bc.reshape(1, _C)

    degp = _deg(edges, ones8, z8)
    xw_pk = _tc_a0(xpad, xpad, W1p)
    dinv2, dinv16, y1_pk = _tc_da(xw_pk, *([degp] * 10))
    y1 = y1_pk.reshape(_NP, _HP1)
    agg1 = _agg1(y1, edges, zrows1)
    agg1_pk = agg1.reshape(_NC, _NH, 128)
    y2_pk = _tc_b(agg1_pk, y1_pk, dinv2, dinv16, b1pk, W2bd)
    y2 = y2_pk.reshape(_NP, _HP2)
    agg2 = _agg2(y2, edges, zrows2)
    agg2_pk = agg2.reshape(_NC, _NQ, 128)
    h, out = _tc_c(agg2_pk, y2_pk, dinv16, b2pk, Wcp, bcp)
    return (out, h)
